# Initial kernel scaffold; baseline (speedup 1.0000x reference)
import jax
import jax.numpy as jnp
from jax.experimental import pallas as pl


def kernel(feats, pts, knn_idx, params):
    raise NotImplementedError("write your pallas kernel here")


# speedup vs baseline: 369.4732x; 369.4732x over previous
import functools

import jax
import jax.numpy as jnp
from jax import lax
from jax.experimental import pallas as pl
from jax.experimental.pallas import tpu as pltpu
from jax.experimental.pallas import tpu_sc as plsc

F32 = jnp.float32

_NC, _NS = 2, 16
_NW = _NC * _NS
_IDXW = 128
_CB = 8


def _sc_gather(table, idx2, width):
    groups = idx2.shape[0]
    rows_per_iter = _CB * _IDXW
    per_w_groups = groups // _NW
    iters = per_w_groups // _CB
    mesh = plsc.VectorSubcoreMesh(core_axis_name="c", subcore_axis_name="s")

    @functools.partial(
        pl.kernel,
        mesh=mesh,
        out_type=jax.ShapeDtypeStruct((groups * _IDXW, width), F32),
        scratch_types=[
            pltpu.VMEM((_CB, _IDXW), jnp.int32),
            pltpu.VMEM((rows_per_iter, width), F32),
            pltpu.SemaphoreType.DMA,
        ],
        compiler_params=pltpu.CompilerParams(use_tc_tiling_on_sc=False),
    )
    def k(table_hbm, idx_hbm, out_hbm, idx_v, rows_v, sem):
        wid = lax.axis_index("s") * _NC + lax.axis_index("c")
        gbase = wid * per_w_groups

        def body(t, carry):
            gb = gbase + t * _CB
            pltpu.sync_copy(idx_hbm.at[pl.ds(gb, _CB)], idx_v)
            handles = [
                pltpu.async_copy(
                    table_hbm.at[idx_v.at[j]],
                    rows_v.at[pl.ds(j * _IDXW, _IDXW)],
                    sem,
                )
                for j in range(_CB)
            ]
            for h in handles:
                h.wait()
            pltpu.sync_copy(rows_v, out_hbm.at[pl.ds(gb * _IDXW, rows_per_iter)])
            return carry

        lax.fori_loop(0, iters, body, 0)

    return k(table, idx2)


def _mlp_table(xs, wbs, wf, consts, nb=1000):
    n = xs[0].shape[0]
    npc = len(xs)

    def body(*refs):
        x_refs = refs[:npc]
        wb_refs = refs[npc:2 * npc]
        wf_ref, c_ref, o_ref = refs[2 * npc], refs[2 * npc + 1], refs[2 * npc + 2]
        acc = c_ref[0:1, :]
        for xr, wr in zip(x_refs, wb_refs):
            acc = acc + jnp.dot(xr[...], wr[...], preferred_element_type=F32)
        new = jnp.maximum(acc, 0.0)
        o_ref[...] = jnp.maximum(
            jnp.dot(new, wf_ref[...], preferred_element_type=F32)
            + c_ref[1:2, :], 0.0)

    in_specs = (
        [pl.BlockSpec((nb, x.shape[1]), lambda i: (i, 0)) for x in xs]
        + [pl.BlockSpec(w.shape, lambda i: (0, 0)) for w in wbs]
        + [pl.BlockSpec(wf.shape, lambda i: (0, 0)),
           pl.BlockSpec((8, 64), lambda i: (0, 0))]
    )
    return pl.pallas_call(
        body,
        grid=(n // nb,),
        in_specs=in_specs,
        out_specs=pl.BlockSpec((nb, 64), lambda i: (i, 0)),
        out_shape=jax.ShapeDtypeStruct((n, 64), F32),
    )(*xs, *wbs, wf, consts)


def _edge1(g1, pg, p8, wd, wp, consts, n, k, nb=400):
    nbk = nb * k

    def body(g_ref, pg_ref, p_ref, wd_ref, wp_ref, c_ref, o_ref, d_ref):
        g = g_ref[...]
        p = p_ref[...]
        pb = jnp.broadcast_to(p[:, None, :], (nb, k, 8)).reshape(nbk, 8)
        delta = pg_ref[...] - pb
        d_ref[...] = delta
        d = jnp.maximum(
            jnp.dot(delta, wd_ref[...], preferred_element_type=F32)
            + c_ref[0:1, :], 0.0)
        h = d * g
        o = jnp.maximum(
            jnp.dot(h, wp_ref[...], preferred_element_type=F32)
            + c_ref[1:2, :], 0.0)
        o_ref[...] = jnp.sum(o.reshape(nb, k, 64), axis=1)

    return pl.pallas_call(
        body,
        grid=(n // nb,),
        in_specs=[
            pl.BlockSpec((nbk, 64), lambda i: (i, 0)),
            pl.BlockSpec((nbk, 8), lambda i: (i, 0)),
            pl.BlockSpec((nb, 8), lambda i: (i, 0)),
            pl.BlockSpec((8, 64), lambda i: (0, 0)),
            pl.BlockSpec((64, 64), lambda i: (0, 0)),
            pl.BlockSpec((8, 64), lambda i: (0, 0)),
        ],
        out_specs=[
            pl.BlockSpec((nb, 64), lambda i: (i, 0)),
            pl.BlockSpec((nbk, 8), lambda i: (i, 0)),
        ],
        out_shape=[
            jax.ShapeDtypeStruct((n, 64), F32),
            jax.ShapeDtypeStruct((g1.shape[0], 8), F32),
        ],
    )(g1, pg, p8, wd, wp, consts)


def _edge(gi, delta, wd, wp, consts, n, k, nb=400):
    nbk = nb * k

    def body(g_ref, d_ref, wd_ref, wp_ref, c_ref, o_ref):
        g = g_ref[...]
        d = jnp.maximum(
            jnp.dot(d_ref[...], wd_ref[...], preferred_element_type=F32)
            + c_ref[0:1, :], 0.0)
        h = d * g
        o = jnp.maximum(
            jnp.dot(h, wp_ref[...], preferred_element_type=F32)
            + c_ref[1:2, :], 0.0)
        o_ref[...] = jnp.sum(o.reshape(nb, k, 64), axis=1)

    return pl.pallas_call(
        body,
        grid=(n // nb,),
        in_specs=[
            pl.BlockSpec((nbk, 64), lambda i: (i, 0)),
            pl.BlockSpec((nbk, 8), lambda i: (i, 0)),
            pl.BlockSpec((8, 64), lambda i: (0, 0)),
            pl.BlockSpec((64, 64), lambda i: (0, 0)),
            pl.BlockSpec((8, 64), lambda i: (0, 0)),
        ],
        out_specs=pl.BlockSpec((nb, 64), lambda i: (i, 0)),
        out_shape=jax.ShapeDtypeStruct((n, 64), F32),
    )(gi, delta, wd, wp, consts)


def _fold(w, b, g, bt):
    return (w * g[:, None]).T, b * g + bt


def _consts(*rows):
    m = jnp.stack(rows)
    return jnp.pad(m, ((0, 8 - m.shape[0]), (0, 0)))


def kernel(feats, pts, knn_idx, params):
    b, fdim, n = feats.shape
    k = knn_idx.shape[2]
    e = n * k
    epad = _NW * _CB * _IDXW * (-(-e // (_NW * _CB * _IDXW)))

    x0 = feats[0].T
    p8 = jnp.pad(pts[0].T, ((0, 0), (0, 5)))
    idx_flat = knn_idx[0].reshape(-1)
    idx2 = jnp.pad(idx_flat, (0, epad - e)).reshape(epad // _IDXW, _IDXW)

    fold = []
    for p in params:
        wb, cb = _fold(p['Wb'], p['bb'], p['gb'], p['betab'])
        wf, cf = _fold(p['Wf'], p['bf'], p['gf'], p['betaf'])
        wd, cd = _fold(p['Wd'], p['bd'], p['gd'], p['betad'])
        wd = jnp.pad(wd, ((0, 5), (0, 0)))
        wp, cp = _fold(p['Wp'], p['bp'], p['gp'], p['betap'])
        fold.append(dict(wb=wb, wf=wf, wd=wd, wp=wp,
                         cab=_consts(cb, cf), ccd=_consts(cd, cp)))

    pg = _sc_gather(p8, idx2, 8)

    f1 = fold[0]
    table1 = _mlp_table([x0], [f1['wb']], f1['wf'], f1['cab'])
    g1 = _sc_gather(table1, idx2, 64)
    o1, delta = _edge1(g1, pg, p8, f1['wd'], f1['wp'], f1['ccd'], n, k)

    f2 = fold[1]
    table2 = _mlp_table([x0, o1], [f2['wb'][:fdim], f2['wb'][fdim:]],
                        f2['wf'], f2['cab'])
    g2 = _sc_gather(table2, idx2, 64)
    o2 = _edge(g2, delta, f2['wd'], f2['wp'], f2['ccd'], n, k)

    f3 = fold[2]
    table3 = _mlp_table(
        [x0, o1, o2],
        [f3['wb'][:fdim], f3['wb'][fdim:fdim + 64], f3['wb'][fdim + 64:]],
        f3['wf'], f3['cab'])
    g3 = _sc_gather(table3, idx2, 64)
    o3 = _edge(g3, delta, f3['wd'], f3['wp'], f3['ccd'], n, k)

    return jnp.concatenate(
        [feats, o1.T[None], o2.T[None], o3.T[None]], axis=1)


# speedup vs baseline: 602.2329x; 1.6300x over previous
import functools

import jax
import jax.numpy as jnp
from jax import lax
from jax.experimental import pallas as pl
from jax.experimental.pallas import tpu as pltpu
from jax.experimental.pallas import tpu_sc as plsc

F32 = jnp.float32

_NC, _NS = 2, 16
_NW = _NC * _NS
_IDXW = 128
_CB = 8


def _sc_gather(table, idx2, width):
    groups = idx2.shape[0]
    nrows = table.shape[0]
    rows_per_iter = _CB * _IDXW
    per_w_groups = groups // _NW
    iters = per_w_groups // _CB
    rows_per_sub = nrows // _NS
    mesh = plsc.VectorSubcoreMesh(core_axis_name="c", subcore_axis_name="s")

    @functools.partial(
        pl.kernel,
        mesh=mesh,
        out_type=jax.ShapeDtypeStruct((groups * _IDXW, width), F32),
        scratch_types=[
            pltpu.VMEM((_CB, _IDXW), jnp.int32),
            pltpu.VMEM((rows_per_iter, width), F32),
            pltpu.VMEM_SHARED((nrows, width), F32),
            pltpu.SemaphoreType.DMA,
        ],
        compiler_params=pltpu.CompilerParams(use_tc_tiling_on_sc=False),
    )
    def k(table_hbm, idx_hbm, out_hbm, idx_v, rows_v, tbl_sh, sem):
        cid = lax.axis_index("c")
        sid = lax.axis_index("s")
        wid = sid * _NC + cid
        gbase = wid * per_w_groups

        pltpu.sync_copy(table_hbm.at[pl.ds(sid * rows_per_sub, rows_per_sub)],
                        tbl_sh.at[pl.ds(sid * rows_per_sub, rows_per_sub)])
        plsc.subcore_barrier()

        def body(t, carry):
            gb = gbase + t * _CB
            pltpu.sync_copy(idx_hbm.at[pl.ds(gb, _CB)], idx_v)
            handles = [
                pltpu.async_copy(
                    tbl_sh.at[idx_v.at[j]],
                    rows_v.at[pl.ds(j * _IDXW, _IDXW)],
                    sem,
                )
                for j in range(_CB)
            ]
            for h in handles:
                h.wait()
            pltpu.sync_copy(rows_v, out_hbm.at[pl.ds(gb * _IDXW, rows_per_iter)])
            return carry

        lax.fori_loop(0, iters, body, 0)

    return k(table, idx2)


def _mlp_table(xs, wbs, wf, consts, nb=1000):
    n = xs[0].shape[0]
    npc = len(xs)

    def body(*refs):
        x_refs = refs[:npc]
        wb_refs = refs[npc:2 * npc]
        wf_ref, c_ref, o_ref = refs[2 * npc], refs[2 * npc + 1], refs[2 * npc + 2]
        acc = c_ref[0:1, :]
        for xr, wr in zip(x_refs, wb_refs):
            acc = acc + jnp.dot(xr[...], wr[...], preferred_element_type=F32)
        new = jnp.maximum(acc, 0.0)
        o_ref[...] = jnp.maximum(
            jnp.dot(new, wf_ref[...], preferred_element_type=F32)
            + c_ref[1:2, :], 0.0)

    in_specs = (
        [pl.BlockSpec((nb, x.shape[1]), lambda i: (i, 0)) for x in xs]
        + [pl.BlockSpec(w.shape, lambda i: (0, 0)) for w in wbs]
        + [pl.BlockSpec(wf.shape, lambda i: (0, 0)),
           pl.BlockSpec((8, 64), lambda i: (0, 0))]
    )
    return pl.pallas_call(
        body,
        grid=(n // nb,),
        in_specs=in_specs,
        out_specs=pl.BlockSpec((nb, 64), lambda i: (i, 0)),
        out_shape=jax.ShapeDtypeStruct((n, 64), F32),
    )(*xs, *wbs, wf, consts)


def _edge1(g1, pg, p8, wd, wp, consts, n, k, nb=400):
    nbk = nb * k

    def body(g_ref, pg_ref, p_ref, wd_ref, wp_ref, c_ref, o_ref, d_ref):
        g = g_ref[...]
        p = p_ref[...]
        pb = jnp.broadcast_to(p[:, None, :], (nb, k, 8)).reshape(nbk, 8)
        delta = pg_ref[...] - pb
        d_ref[...] = delta
        d = jnp.maximum(
            jnp.dot(delta, wd_ref[...], preferred_element_type=F32)
            + c_ref[0:1, :], 0.0)
        h = d * g
        o = jnp.maximum(
            jnp.dot(h, wp_ref[...], preferred_element_type=F32)
            + c_ref[1:2, :], 0.0)
        o_ref[...] = jnp.sum(o.reshape(nb, k, 64), axis=1)

    return pl.pallas_call(
        body,
        grid=(n // nb,),
        in_specs=[
            pl.BlockSpec((nbk, 64), lambda i: (i, 0)),
            pl.BlockSpec((nbk, 8), lambda i: (i, 0)),
            pl.BlockSpec((nb, 8), lambda i: (i, 0)),
            pl.BlockSpec((8, 64), lambda i: (0, 0)),
            pl.BlockSpec((64, 64), lambda i: (0, 0)),
            pl.BlockSpec((8, 64), lambda i: (0, 0)),
        ],
        out_specs=[
            pl.BlockSpec((nb, 64), lambda i: (i, 0)),
            pl.BlockSpec((nbk, 8), lambda i: (i, 0)),
        ],
        out_shape=[
            jax.ShapeDtypeStruct((n, 64), F32),
            jax.ShapeDtypeStruct((g1.shape[0], 8), F32),
        ],
    )(g1, pg, p8, wd, wp, consts)


def _edge(gi, delta, wd, wp, consts, n, k, nb=400):
    nbk = nb * k

    def body(g_ref, d_ref, wd_ref, wp_ref, c_ref, o_ref):
        g = g_ref[...]
        d = jnp.maximum(
            jnp.dot(d_ref[...], wd_ref[...], preferred_element_type=F32)
            + c_ref[0:1, :], 0.0)
        h = d * g
        o = jnp.maximum(
            jnp.dot(h, wp_ref[...], preferred_element_type=F32)
            + c_ref[1:2, :], 0.0)
        o_ref[...] = jnp.sum(o.reshape(nb, k, 64), axis=1)

    return pl.pallas_call(
        body,
        grid=(n // nb,),
        in_specs=[
            pl.BlockSpec((nbk, 64), lambda i: (i, 0)),
            pl.BlockSpec((nbk, 8), lambda i: (i, 0)),
            pl.BlockSpec((8, 64), lambda i: (0, 0)),
            pl.BlockSpec((64, 64), lambda i: (0, 0)),
            pl.BlockSpec((8, 64), lambda i: (0, 0)),
        ],
        out_specs=pl.BlockSpec((nb, 64), lambda i: (i, 0)),
        out_shape=jax.ShapeDtypeStruct((n, 64), F32),
    )(gi, delta, wd, wp, consts)


def _fold(w, b, g, bt):
    return (w * g[:, None]).T, b * g + bt


def _consts(*rows):
    m = jnp.stack(rows)
    return jnp.pad(m, ((0, 8 - m.shape[0]), (0, 0)))


def kernel(feats, pts, knn_idx, params):
    b, fdim, n = feats.shape
    k = knn_idx.shape[2]
    e = n * k
    epad = _NW * _CB * _IDXW * (-(-e // (_NW * _CB * _IDXW)))

    x0 = feats[0].T
    p8 = jnp.pad(pts[0].T, ((0, 0), (0, 5)))
    idx_flat = knn_idx[0].reshape(-1)
    idx2 = jnp.pad(idx_flat, (0, epad - e)).reshape(epad // _IDXW, _IDXW)

    fold = []
    for p in params:
        wb, cb = _fold(p['Wb'], p['bb'], p['gb'], p['betab'])
        wf, cf = _fold(p['Wf'], p['bf'], p['gf'], p['betaf'])
        wd, cd = _fold(p['Wd'], p['bd'], p['gd'], p['betad'])
        wd = jnp.pad(wd, ((0, 5), (0, 0)))
        wp, cp = _fold(p['Wp'], p['bp'], p['gp'], p['betap'])
        fold.append(dict(wb=wb, wf=wf, wd=wd, wp=wp,
                         cab=_consts(cb, cf), ccd=_consts(cd, cp)))

    pg = _sc_gather(p8, idx2, 8)

    f1 = fold[0]
    table1 = _mlp_table([x0], [f1['wb']], f1['wf'], f1['cab'])
    g1 = _sc_gather(table1, idx2, 64)
    o1, delta = _edge1(g1, pg, p8, f1['wd'], f1['wp'], f1['ccd'], n, k)

    f2 = fold[1]
    table2 = _mlp_table([x0, o1], [f2['wb'][:fdim], f2['wb'][fdim:]],
                        f2['wf'], f2['cab'])
    g2 = _sc_gather(table2, idx2, 64)
    o2 = _edge(g2, delta, f2['wd'], f2['wp'], f2['ccd'], n, k)

    f3 = fold[2]
    table3 = _mlp_table(
        [x0, o1, o2],
        [f3['wb'][:fdim], f3['wb'][fdim:fdim + 64], f3['wb'][fdim + 64:]],
        f3['wf'], f3['cab'])
    g3 = _sc_gather(table3, idx2, 64)
    o3 = _edge(g3, delta, f3['wd'], f3['wp'], f3['ccd'], n, k)

    return jnp.concatenate(
        [feats, o1.T[None], o2.T[None], o3.T[None]], axis=1)


# speedup vs baseline: 633.9466x; 1.0527x over previous
import functools

import jax
import jax.numpy as jnp
from jax import lax
from jax.experimental import pallas as pl
from jax.experimental.pallas import tpu as pltpu
from jax.experimental.pallas import tpu_sc as plsc

F32 = jnp.float32

_NC, _NS = 2, 16
_NW = _NC * _NS
_IDXW = 128
_CB = 8


def _sc_gather(table, idx2, width):
    groups = idx2.shape[0]
    nrows = table.shape[0]
    rows_per_iter = _CB * _IDXW
    per_w_groups = groups // _NW
    iters = per_w_groups // _CB
    rows_per_sub = nrows // _NS
    mesh = plsc.VectorSubcoreMesh(core_axis_name="c", subcore_axis_name="s")

    @functools.partial(
        pl.kernel,
        mesh=mesh,
        out_type=jax.ShapeDtypeStruct((groups * _IDXW, width), F32),
        scratch_types=[
            pltpu.VMEM((_CB, _IDXW), jnp.int32),
            pltpu.VMEM((rows_per_iter, width), F32),
            pltpu.VMEM_SHARED((nrows, width), F32),
            pltpu.SemaphoreType.DMA,
        ],
        compiler_params=pltpu.CompilerParams(use_tc_tiling_on_sc=False),
    )
    def k(table_hbm, idx_hbm, out_hbm, idx_v, rows_v, tbl_sh, sem):
        cid = lax.axis_index("c")
        sid = lax.axis_index("s")
        wid = sid * _NC + cid
        gbase = wid * per_w_groups

        pltpu.sync_copy(table_hbm.at[pl.ds(sid * rows_per_sub, rows_per_sub)],
                        tbl_sh.at[pl.ds(sid * rows_per_sub, rows_per_sub)])
        plsc.subcore_barrier()

        def body(t, carry):
            gb = gbase + t * _CB
            pltpu.sync_copy(idx_hbm.at[pl.ds(gb, _CB)], idx_v)
            handles = [
                pltpu.async_copy(
                    tbl_sh.at[idx_v.at[j]],
                    rows_v.at[pl.ds(j * _IDXW, _IDXW)],
                    sem,
                )
                for j in range(_CB)
            ]
            for h in handles:
                h.wait()
            pltpu.sync_copy(rows_v, out_hbm.at[pl.ds(gb * _IDXW, rows_per_iter)])
            return carry

        lax.fori_loop(0, iters, body, 0)

    return k(table, idx2)


def _mlp_table(xs, wbs, wf, consts, nb=1024):
    n = xs[0].shape[1]
    npc = len(xs)

    def body(*refs):
        x_refs = refs[:npc]
        wb_refs = refs[npc:2 * npc]
        wf_ref, c_ref, o_ref = refs[2 * npc], refs[2 * npc + 1], refs[2 * npc + 2]
        acc = c_ref[:, 0:1]
        for xr, wr in zip(x_refs, wb_refs):
            acc = acc + jnp.dot(wr[...], xr[...], preferred_element_type=F32)
        new = jnp.maximum(acc, 0.0)
        f = jnp.maximum(
            jnp.dot(wf_ref[...], new, preferred_element_type=F32)
            + c_ref[:, 1:2], 0.0)
        o_ref[...] = f.T

    in_specs = (
        [pl.BlockSpec((x.shape[0], nb), lambda i: (0, i)) for x in xs]
        + [pl.BlockSpec(w.shape, lambda i: (0, 0)) for w in wbs]
        + [pl.BlockSpec(wf.shape, lambda i: (0, 0)),
           pl.BlockSpec((64, 8), lambda i: (0, 0))]
    )
    return pl.pallas_call(
        body,
        grid=(-(-n // nb),),
        in_specs=in_specs,
        out_specs=pl.BlockSpec((nb, 64), lambda i: (i, 0)),
        out_shape=jax.ShapeDtypeStruct((n, 64), F32),
    )(*xs, *wbs, wf, consts)


def _edge(gi, pg, p8, wd, wp, consts, n, k, nb=512):
    nbk = nb * k

    def body(g_ref, pg_ref, p_ref, wd_ref, wp_ref, c_ref, o_ref):
        g = g_ref[...]
        p = p_ref[...]
        pb = jnp.broadcast_to(p[:, None, :], (nb, k, 8)).reshape(nbk, 8)
        delta = pg_ref[...] - pb
        d = jnp.maximum(
            jnp.dot(delta, wd_ref[...], preferred_element_type=F32)
            + c_ref[0:1, :], 0.0)
        h = d * g
        o = jnp.maximum(
            jnp.dot(h, wp_ref[...], preferred_element_type=F32)
            + c_ref[1:2, :], 0.0)
        o_ref[...] = jnp.sum(o.reshape(nb, k, 64), axis=1).T

    return pl.pallas_call(
        body,
        grid=(-(-n // nb),),
        in_specs=[
            pl.BlockSpec((nbk, 64), lambda i: (i, 0)),
            pl.BlockSpec((nbk, 8), lambda i: (i, 0)),
            pl.BlockSpec((nb, 8), lambda i: (i, 0)),
            pl.BlockSpec((8, 64), lambda i: (0, 0)),
            pl.BlockSpec((64, 64), lambda i: (0, 0)),
            pl.BlockSpec((8, 64), lambda i: (0, 0)),
        ],
        out_specs=pl.BlockSpec((64, nb), lambda i: (0, i)),
        out_shape=jax.ShapeDtypeStruct((64, n), F32),
    )(gi, pg, p8, wd, wp, consts)


def _fold(w, b, g, bt):
    return w * g[:, None], b * g + bt


def _consts_rows(*rows):
    m = jnp.stack(rows)
    return jnp.pad(m, ((0, 8 - m.shape[0]), (0, 0)))


def kernel(feats, pts, knn_idx, params):
    b, fdim, n = feats.shape
    k = knn_idx.shape[2]
    e = n * k
    epad = _NW * _CB * _IDXW * (-(-e // (_NW * _CB * _IDXW)))

    x0 = feats[0]
    p8 = jnp.pad(pts[0].T, ((0, 0), (0, 5)))
    idx_flat = knn_idx[0].reshape(-1)
    idx2 = jnp.pad(idx_flat, (0, epad - e)).reshape(epad // _IDXW, _IDXW)

    fold = []
    for p in params:
        wb, cb = _fold(p['Wb'], p['bb'], p['gb'], p['betab'])
        wf, cf = _fold(p['Wf'], p['bf'], p['gf'], p['betaf'])
        wd, cd = _fold(p['Wd'], p['bd'], p['gd'], p['betad'])
        wd = jnp.pad(wd.T, ((0, 5), (0, 0)))
        wp, cp = _fold(p['Wp'], p['bp'], p['gp'], p['betap'])
        fold.append(dict(
            wb=wb, wf=wf, wd=wd, wp=wp.T,
            cab=jnp.stack([cb, cf], axis=1),
            ccd=_consts_rows(cd, cp)))
    for f in fold:
        f['cab'] = jnp.pad(f['cab'], ((0, 0), (0, 6)))

    pg = _sc_gather(p8, idx2, 8)

    f1 = fold[0]
    table1 = _mlp_table([x0], [f1['wb']], f1['wf'], f1['cab'])
    g1 = _sc_gather(table1, idx2, 64)
    o1 = _edge(g1, pg, p8, f1['wd'], f1['wp'], f1['ccd'], n, k)

    f2 = fold[1]
    table2 = _mlp_table([x0, o1], [f2['wb'][:, :fdim], f2['wb'][:, fdim:]],
                        f2['wf'], f2['cab'])
    g2 = _sc_gather(table2, idx2, 64)
    o2 = _edge(g2, pg, p8, f2['wd'], f2['wp'], f2['ccd'], n, k)

    f3 = fold[2]
    table3 = _mlp_table(
        [x0, o1, o2],
        [f3['wb'][:, :fdim], f3['wb'][:, fdim:fdim + 64],
         f3['wb'][:, fdim + 64:]],
        f3['wf'], f3['cab'])
    g3 = _sc_gather(table3, idx2, 64)
    o3 = _edge(g3, pg, p8, f3['wd'], f3['wp'], f3['ccd'], n, k)

    return jnp.concatenate(
        [feats, o1[None], o2[None], o3[None]], axis=1)


# speedup vs baseline: 939.9282x; 1.4827x over previous
import functools

import jax
import jax.numpy as jnp
from jax import lax
from jax.experimental import pallas as pl
from jax.experimental.pallas import tpu as pltpu
from jax.experimental.pallas import tpu_sc as plsc

F32 = jnp.float32

_NC, _NS = 2, 16
_NW = _NC * _NS
_IDXW = 128
_CB = 2
_TW = 128


def _sc_gather(table, idx2):
    groups = idx2.shape[0]
    nrows = table.shape[0]
    rows_per_iter = _CB * _IDXW
    per_w_groups = groups // _NW
    iters = per_w_groups // _CB
    rows_per_sub = nrows // _NS
    mesh = plsc.VectorSubcoreMesh(core_axis_name="c", subcore_axis_name="s")

    @functools.partial(
        pl.kernel,
        mesh=mesh,
        out_type=jax.ShapeDtypeStruct((groups * _IDXW, _TW), F32),
        scratch_types=[
            pltpu.VMEM((_CB, _IDXW), jnp.int32),
            pltpu.VMEM((rows_per_iter, _TW), F32),
            pltpu.VMEM_SHARED((nrows, _TW), F32),
            pltpu.SemaphoreType.DMA,
        ],
        compiler_params=pltpu.CompilerParams(use_tc_tiling_on_sc=False),
    )
    def k(table_hbm, idx_hbm, out_hbm, idx_v, rows_v, tbl_sh, sem):
        cid = lax.axis_index("c")
        sid = lax.axis_index("s")
        wid = sid * _NC + cid
        gbase = wid * per_w_groups

        pltpu.sync_copy(table_hbm.at[pl.ds(sid * rows_per_sub, rows_per_sub)],
                        tbl_sh.at[pl.ds(sid * rows_per_sub, rows_per_sub)])
        plsc.subcore_barrier()

        def body(t, carry):
            gb = gbase + t * _CB
            pltpu.sync_copy(idx_hbm.at[pl.ds(gb, _CB)], idx_v)
            handles = [
                pltpu.async_copy(
                    tbl_sh.at[idx_v.at[j]],
                    rows_v.at[pl.ds(j * _IDXW, _IDXW)],
                    sem,
                )
                for j in range(_CB)
            ]
            for h in handles:
                h.wait()
            pltpu.sync_copy(rows_v, out_hbm.at[pl.ds(gb * _IDXW, rows_per_iter)])
            return carry

        lax.fori_loop(0, iters, body, 0)

    return k(table, idx2)


def _mlp_table(xs, p3, wbs, wf, wq, consts, nb=1024):
    n = xs[0].shape[1]
    npc = len(xs)

    def body(*refs):
        x_refs = refs[:npc]
        p_ref = refs[npc]
        wb_refs = refs[npc + 1:2 * npc + 1]
        wf_ref, wq_ref, c_ref, o_ref = refs[2 * npc + 1:]
        acc = c_ref[:, 0:1]
        for xr, wr in zip(x_refs, wb_refs):
            acc = acc + jnp.dot(wr[...], xr[...], preferred_element_type=F32)
        new = jnp.maximum(acc, 0.0)
        f = jnp.maximum(
            jnp.dot(wf_ref[...], new, preferred_element_type=F32)
            + c_ref[:, 1:2], 0.0)
        q = jnp.dot(wq_ref[...], p_ref[...], preferred_element_type=F32)
        o_ref[...] = jnp.concatenate([f, q], axis=0).T

    in_specs = (
        [pl.BlockSpec((x.shape[0], nb), lambda i: (0, i)) for x in xs]
        + [pl.BlockSpec((3, nb), lambda i: (0, i))]
        + [pl.BlockSpec(w.shape, lambda i: (0, 0)) for w in wbs]
        + [pl.BlockSpec(wf.shape, lambda i: (0, 0)),
           pl.BlockSpec(wq.shape, lambda i: (0, 0)),
           pl.BlockSpec((64, 8), lambda i: (0, 0))]
    )
    return pl.pallas_call(
        body,
        grid=(-(-n // nb),),
        in_specs=in_specs,
        out_specs=pl.BlockSpec((nb, _TW), lambda i: (i, 0)),
        out_shape=jax.ShapeDtypeStruct((n, _TW), F32),
    )(*xs, p3, *wbs, wf, wq, consts)


def _edge(g3, table, wp, consts, n, k, nb=512):
    def body(g_ref, t_ref, wp_ref, c_ref, o_ref):
        lane = lax.broadcasted_iota(jnp.int32, (1, _TW), 1)
        qmask = jnp.where(lane >= 64, 1.0, 0.0)
        s = t_ref[...] * qmask - c_ref[0:1, :]
        cp = c_ref[1:2, :64]
        wp2 = wp_ref[...]
        acc = jnp.zeros((nb, 64), F32)
        for kk in range(k):
            u = jnp.maximum(g_ref[kk] - s, 0.0)
            w = u * pltpu.roll(u, 64, 1)
            acc = acc + jnp.maximum(
                jnp.dot(w, wp2, preferred_element_type=F32) + cp, 0.0)
        o_ref[...] = acc.T

    return pl.pallas_call(
        body,
        grid=(-(-n // nb),),
        in_specs=[
            pl.BlockSpec((k, nb, _TW), lambda i: (0, i, 0)),
            pl.BlockSpec((nb, _TW), lambda i: (i, 0)),
            pl.BlockSpec((_TW, 64), lambda i: (0, 0)),
            pl.BlockSpec((8, _TW), lambda i: (0, 0)),
        ],
        out_specs=pl.BlockSpec((64, nb), lambda i: (0, i)),
        out_shape=jax.ShapeDtypeStruct((64, n), F32),
    )(g3, table, wp, consts)


def _fold(w, b, g, bt):
    return w * g[:, None], b * g + bt


def _consts_rows(*rows):
    m = jnp.stack(rows)
    return jnp.pad(m, ((0, 8 - m.shape[0]), (0, 0)))


def kernel(feats, pts, knn_idx, params):
    b, fdim, n = feats.shape
    k = knn_idx.shape[2]
    e = n * k
    chunk = _NW * _CB * _IDXW
    epad = chunk * (-(-e // chunk))

    x0 = feats[0]
    p3 = pts[0]
    npad = epad // k
    idx_km = jnp.pad(knn_idx[0].T, ((0, 0), (0, npad - n)))
    idx2 = idx_km.reshape(epad // _IDXW, _IDXW)

    fold = []
    for p in params:
        wb, cb = _fold(p['Wb'], p['bb'], p['gb'], p['betab'])
        wf, cf = _fold(p['Wf'], p['bf'], p['gf'], p['betaf'])
        wq, cd = _fold(p['Wd'], p['bd'], p['gd'], p['betad'])
        wp, cp = _fold(p['Wp'], p['bp'], p['gp'], p['betap'])
        z64 = jnp.zeros((64,), F32)
        fold.append(dict(
            wb=wb, wf=wf, wq=wq,
            wp=jnp.pad(wp.T, ((0, 64), (0, 0))),
            cab=jnp.pad(jnp.stack([cb, cf], axis=1), ((0, 0), (0, 6))),
            ccd=_consts_rows(jnp.concatenate([z64, cd]),
                             jnp.concatenate([cp, z64]))))

    outs = []
    xpieces, wsplit = [x0], [fdim]
    for i, f in enumerate(fold):
        splits = []
        off = 0
        for w in wsplit:
            splits.append((off, w))
            off += w
        wbs = [f['wb'][:, o:o + w] for (o, w) in splits]
        table = _mlp_table(xpieces, p3, wbs, f['wf'], f['wq'], f['cab'])
        g3 = _sc_gather(table, idx2).reshape(k, npad, _TW)
        oi = _edge(g3, table, f['wp'], f['ccd'], n, k)
        outs.append(oi)
        xpieces = xpieces + [oi]
        wsplit = wsplit + [64]

    return jnp.concatenate([feats] + [o[None] for o in outs], axis=1)


# speedup vs baseline: 1041.2600x; 1.1078x over previous
import functools

import jax
import jax.numpy as jnp
from jax import lax
from jax.experimental import pallas as pl
from jax.experimental.pallas import tpu as pltpu
from jax.experimental.pallas import tpu_sc as plsc

F32 = jnp.float32

_NC, _NS = 2, 16
_NW = _NC * _NS
_IDXW = 128
_CB = 2
_TW = 128


def _sc_gather(table, idx2):
    groups = idx2.shape[0]
    nrows = table.shape[0]
    per_w_groups = groups // _NW
    iters = per_w_groups // 2
    rows_per_sub = nrows // _NS
    mesh = plsc.VectorSubcoreMesh(core_axis_name="c", subcore_axis_name="s")

    @functools.partial(
        pl.kernel,
        mesh=mesh,
        out_type=jax.ShapeDtypeStruct((groups * _IDXW, _TW), F32),
        scratch_types=[
            pltpu.VMEM((per_w_groups, _IDXW), jnp.int32),
            pltpu.VMEM((2, _IDXW, _TW), F32),
            pltpu.VMEM_SHARED((nrows, _TW), F32),
            pltpu.SemaphoreType.DMA,
            pltpu.SemaphoreType.DMA,
            pltpu.SemaphoreType.DMA,
            pltpu.SemaphoreType.DMA,
        ],
        compiler_params=pltpu.CompilerParams(use_tc_tiling_on_sc=False),
    )
    def k(table_hbm, idx_hbm, out_hbm, idx_v, rows_v, tbl_sh,
          gs0, gs1, ss0, ss1):
        cid = lax.axis_index("c")
        sid = lax.axis_index("s")
        wid = sid * _NC + cid
        gbase = wid * per_w_groups

        pltpu.sync_copy(idx_hbm.at[pl.ds(gbase, per_w_groups)], idx_v)
        pltpu.sync_copy(table_hbm.at[pl.ds(sid * rows_per_sub, rows_per_sub)],
                        tbl_sh.at[pl.ds(sid * rows_per_sub, rows_per_sub)])
        plsc.subcore_barrier()

        gsems = (gs0, gs1)
        ssems = (ss0, ss1)

        def body(t, carry):
            c0 = 2 * t
            gh = []
            for bb in range(2):
                gh.append(pltpu.async_copy(
                    tbl_sh.at[idx_v.at[c0 + bb]], rows_v.at[bb], gsems[bb]))
            sh = []
            for bb in range(2):
                gh[bb].wait()
                sh.append(pltpu.async_copy(
                    rows_v.at[bb],
                    out_hbm.at[pl.ds((gbase + c0 + bb) * _IDXW, _IDXW)],
                    ssems[bb]))
            for bb in range(2):
                sh[bb].wait()
            return carry

        lax.fori_loop(0, iters, body, 0)

    return k(table, idx2)


def _mlp_table(xs, p3, wbs, wf, wq, consts, nb=1024):
    n = xs[0].shape[1]
    npc = len(xs)

    def body(*refs):
        x_refs = refs[:npc]
        p_ref = refs[npc]
        wb_refs = refs[npc + 1:2 * npc + 1]
        wf_ref, wq_ref, c_ref, o_ref = refs[2 * npc + 1:]
        acc = c_ref[:, 0:1]
        for xr, wr in zip(x_refs, wb_refs):
            acc = acc + jnp.dot(wr[...], xr[...], preferred_element_type=F32)
        new = jnp.maximum(acc, 0.0)
        f = jnp.maximum(
            jnp.dot(wf_ref[...], new, preferred_element_type=F32)
            + c_ref[:, 1:2], 0.0)
        q = jnp.dot(wq_ref[...], p_ref[...], preferred_element_type=F32)
        o_ref[...] = jnp.concatenate([f, q], axis=0).T

    in_specs = (
        [pl.BlockSpec((x.shape[0], nb), lambda i: (0, i)) for x in xs]
        + [pl.BlockSpec((3, nb), lambda i: (0, i))]
        + [pl.BlockSpec(w.shape, lambda i: (0, 0)) for w in wbs]
        + [pl.BlockSpec(wf.shape, lambda i: (0, 0)),
           pl.BlockSpec(wq.shape, lambda i: (0, 0)),
           pl.BlockSpec((64, 8), lambda i: (0, 0))]
    )
    return pl.pallas_call(
        body,
        grid=(-(-n // nb),),
        in_specs=in_specs,
        out_specs=pl.BlockSpec((nb, _TW), lambda i: (i, 0)),
        out_shape=jax.ShapeDtypeStruct((n, _TW), F32),
    )(*xs, p3, *wbs, wf, wq, consts)


def _edge(g3, table, wp, consts, n, k, nb=512):
    def body(g_ref, t_ref, wp_ref, c_ref, o_ref):
        lane = lax.broadcasted_iota(jnp.int32, (1, _TW), 1)
        qmask = jnp.where(lane >= 64, 1.0, 0.0)
        s = t_ref[...] * qmask - c_ref[0:1, :]
        cp = c_ref[1:2, :64]
        wp2 = wp_ref[...]
        acc = jnp.zeros((nb, 64), F32)
        for kk in range(k):
            u = jnp.maximum(g_ref[kk] - s, 0.0)
            w = u * pltpu.roll(u, 64, 1)
            acc = acc + jnp.maximum(
                jnp.dot(w, wp2, preferred_element_type=F32) + cp, 0.0)
        o_ref[...] = acc.T

    return pl.pallas_call(
        body,
        grid=(-(-n // nb),),
        in_specs=[
            pl.BlockSpec((k, nb, _TW), lambda i: (0, i, 0)),
            pl.BlockSpec((nb, _TW), lambda i: (i, 0)),
            pl.BlockSpec((_TW, 64), lambda i: (0, 0)),
            pl.BlockSpec((8, _TW), lambda i: (0, 0)),
        ],
        out_specs=pl.BlockSpec((64, nb), lambda i: (0, i)),
        out_shape=jax.ShapeDtypeStruct((64, n), F32),
    )(g3, table, wp, consts)


def _fold(w, b, g, bt):
    return w * g[:, None], b * g + bt


def _consts_rows(*rows):
    m = jnp.stack(rows)
    return jnp.pad(m, ((0, 8 - m.shape[0]), (0, 0)))


def kernel(feats, pts, knn_idx, params):
    b, fdim, n = feats.shape
    k = knn_idx.shape[2]
    e = n * k
    chunk = _NW * _CB * _IDXW
    epad = chunk * (-(-e // chunk))

    x0 = feats[0]
    p3 = pts[0]
    npad = epad // k
    idx_km = jnp.pad(knn_idx[0].T, ((0, 0), (0, npad - n)))
    idx2 = idx_km.reshape(epad // _IDXW, _IDXW)

    fold = []
    for p in params:
        wb, cb = _fold(p['Wb'], p['bb'], p['gb'], p['betab'])
        wf, cf = _fold(p['Wf'], p['bf'], p['gf'], p['betaf'])
        wq, cd = _fold(p['Wd'], p['bd'], p['gd'], p['betad'])
        wp, cp = _fold(p['Wp'], p['bp'], p['gp'], p['betap'])
        z64 = jnp.zeros((64,), F32)
        fold.append(dict(
            wb=wb, wf=wf, wq=wq,
            wp=jnp.pad(wp.T, ((0, 64), (0, 0))),
            cab=jnp.pad(jnp.stack([cb, cf], axis=1), ((0, 0), (0, 6))),
            ccd=_consts_rows(jnp.concatenate([z64, cd]),
                             jnp.concatenate([cp, z64]))))

    outs = []
    xpieces, wsplit = [x0], [fdim]
    for i, f in enumerate(fold):
        splits = []
        off = 0
        for w in wsplit:
            splits.append((off, w))
            off += w
        wbs = [f['wb'][:, o:o + w] for (o, w) in splits]
        table = _mlp_table(xpieces, p3, wbs, f['wf'], f['wq'], f['cab'])
        g3 = _sc_gather(table, idx2).reshape(k, npad, _TW)
        oi = _edge(g3, table, f['wp'], f['ccd'], n, k)
        outs.append(oi)
        xpieces = xpieces + [oi]
        wsplit = wsplit + [64]

    return jnp.concatenate([feats] + [o[None] for o in outs], axis=1)
